# Initial kernel scaffold; baseline (speedup 1.0000x reference)
import functools

import jax
import jax.numpy as jnp
from jax import lax
from jax.experimental import pallas as pl
from jax.experimental.pallas import tpu as pltpu
from jax.experimental.pallas import tpu_sc as plsc

N = 10000
D = 128
H = 128

NW = 32
K = 128
NPAD = 10240
RPW = NPAD // 16
DEGW = 16


def _deg_kernel(dst_hbm, ones_hbm, zeros_hbm, out_hbm, dst_v, ones_v, sem,
                acc_sh, *, n_chunks):
    c = lax.axis_index("c")
    s = lax.axis_index("s")
    wid = c * 16 + s
    pltpu.sync_copy(ones_hbm, ones_v)
    pltpu.sync_copy(zeros_hbm.at[:, :DEGW], acc_sh.at[pl.ds(s * RPW, RPW)])
    plsc.subcore_barrier()

    base = wid * (n_chunks * K)

    def body(i, carry):
        off = base + i * K
        pltpu.sync_copy(dst_hbm.at[pl.ds(off, K)], dst_v)
        pltpu.sync_copy(ones_v, acc_sh.at[dst_v], add=True)
        return carry

    lax.fori_loop(0, n_chunks, body, 0)
    plsc.subcore_barrier()
    pltpu.sync_copy(acc_sh.at[pl.ds(s * RPW, RPW)],
                    out_hbm.at[c, pl.ds(s * RPW, RPW)])


def _agg_kernel(h_hbm, src_hbm, dst_hbm, zeros_hbm, out_hbm,
                src_v, dst_v, rows_v, sem, acc_sh, *, n_chunks):
    c = lax.axis_index("c")
    s = lax.axis_index("s")
    wid = c * 16 + s
    pltpu.sync_copy(zeros_hbm, acc_sh.at[pl.ds(s * RPW, RPW)])
    plsc.subcore_barrier()

    base = wid * (n_chunks * K)

    def body(i, carry):
        off = base + i * K
        pltpu.sync_copy(src_hbm.at[pl.ds(off, K)], src_v)
        pltpu.sync_copy(dst_hbm.at[pl.ds(off, K)], dst_v)
        pltpu.async_copy(h_hbm.at[src_v], rows_v, sem).wait()
        pltpu.sync_copy(rows_v, acc_sh.at[dst_v], add=True)
        return carry

    lax.fori_loop(0, n_chunks, body, 0)
    plsc.subcore_barrier()
    pltpu.sync_copy(acc_sh.at[pl.ds(s * RPW, RPW)],
                    out_hbm.at[c, pl.ds(s * RPW, RPW)])


def _make_deg(n_chunks):
    mesh = plsc.VectorSubcoreMesh(core_axis_name="c", subcore_axis_name="s")
    return pl.kernel(
        functools.partial(_deg_kernel, n_chunks=n_chunks),
        out_type=jax.ShapeDtypeStruct((2, NPAD, DEGW), jnp.float32),
        mesh=mesh,
        scratch_types=[
            pltpu.VMEM((K,), jnp.int32),
            pltpu.VMEM((K, DEGW), jnp.float32),
            pltpu.SemaphoreType.DMA,
            pltpu.MemorySpace.VMEM_SHARED((NPAD, DEGW), jnp.float32),
        ],
    )


def _make_agg(n_chunks):
    mesh = plsc.VectorSubcoreMesh(core_axis_name="c", subcore_axis_name="s")
    return pl.kernel(
        functools.partial(_agg_kernel, n_chunks=n_chunks),
        out_type=jax.ShapeDtypeStruct((2, NPAD, H), jnp.float32),
        mesh=mesh,
        scratch_types=[
            pltpu.VMEM((K,), jnp.int32),
            pltpu.VMEM((K,), jnp.int32),
            pltpu.VMEM((K, H), jnp.float32),
            pltpu.SemaphoreType.DMA,
            pltpu.MemorySpace.VMEM_SHARED((NPAD, H), jnp.float32),
        ],
    )


RB = 2000


def _dinv(dega_ref, degb_ref):
    deg = dega_ref[:, 0:1] + degb_ref[:, 0:1] + 1.0
    return lax.rsqrt(deg)


def _mm_scale_body(x_ref, w_ref, dega_ref, degb_ref, o_ref):
    dinv = _dinv(dega_ref, degb_ref)
    h = jnp.dot(x_ref[...], w_ref[...], preferred_element_type=jnp.float32)
    o_ref[...] = h * dinv


def _mid_body(agg_ref, hp_ref, w_ref, b_ref, dega_ref, degb_ref, o_ref):
    dinv = _dinv(dega_ref, degb_ref)
    tot = agg_ref[0] + agg_ref[1] + hp_ref[...]
    out1 = jnp.maximum(tot * dinv + b_ref[...], 0.0)
    o_ref[...] = jnp.dot(out1, w_ref[...],
                         preferred_element_type=jnp.float32) * dinv


def _final_body(agg_ref, hp_ref, b_ref, dega_ref, degb_ref, o_ref):
    dinv = _dinv(dega_ref, degb_ref)
    tot = agg_ref[0] + agg_ref[1] + hp_ref[...]
    o_ref[...] = jnp.maximum(tot * dinv + b_ref[...], 0.0)


def _row_spec(width):
    return pl.BlockSpec((RB, width), lambda i: (i, 0))


def _full_spec(shape):
    return pl.BlockSpec(shape, lambda i: tuple(0 for _ in shape))


def _mm_scale(x, w, dega, degb):
    return pl.pallas_call(
        _mm_scale_body,
        grid=(N // RB,),
        in_specs=[
            _row_spec(D),
            _full_spec((D, H)),
            _row_spec(DEGW),
            _row_spec(DEGW),
        ],
        out_specs=_row_spec(H),
        out_shape=jax.ShapeDtypeStruct((N, H), jnp.float32),
    )(x, w, dega, degb)


def _mid(agg, hp, w, b, dega, degb):
    return pl.pallas_call(
        _mid_body,
        grid=(N // RB,),
        in_specs=[
            pl.BlockSpec((2, RB, H), lambda i: (0, i, 0)),
            _row_spec(H),
            _full_spec((H, H)),
            _full_spec((1, H)),
            _row_spec(DEGW),
            _row_spec(DEGW),
        ],
        out_specs=_row_spec(H),
        out_shape=jax.ShapeDtypeStruct((N, H), jnp.float32),
    )(agg, hp, w, b, dega, degb)


def _final(agg, hp, b, dega, degb):
    return pl.pallas_call(
        _final_body,
        grid=(N // RB,),
        in_specs=[
            pl.BlockSpec((2, RB, H), lambda i: (0, i, 0)),
            _row_spec(H),
            _full_spec((1, H)),
            _row_spec(DEGW),
            _row_spec(DEGW),
        ],
        out_specs=_row_spec(H),
        out_shape=jax.ShapeDtypeStruct((N, H), jnp.float32),
    )(agg, hp, b, dega, degb)


def kernel(x, edge_index, W1, b1, W2, b2):
    src = edge_index[0]
    dst = edge_index[1]
    e = src.shape[0]

    epad = -(-e // (NW * K)) * (NW * K)
    n_chunks = epad // (NW * K)
    pad = epad - e
    src_p = jnp.concatenate([src, jnp.zeros((pad,), jnp.int32)])
    dst_p = jnp.concatenate([dst, jnp.full((pad,), N, jnp.int32)])

    zeros_deg = jnp.zeros((RPW, H), jnp.float32)
    ones_v = jnp.ones((K, DEGW), jnp.float32)

    deg = _make_deg(n_chunks)(dst_p, ones_v, zeros_deg)
    dega = deg[0]
    degb = deg[1]

    b1r = b1.reshape(1, H)
    b2r = b2.reshape(1, H)

    h1p = _mm_scale(x, W1, dega[:N], degb[:N])
    agg1 = _make_agg(n_chunks)(h1p, src_p, dst_p, zeros_deg)
    h2p = _mid(agg1[:, :N], h1p, W2, b1r, dega[:N], degb[:N])
    agg2 = _make_agg(n_chunks)(h2p, src_p, dst_p, zeros_deg)
    out = _final(agg2[:, :N], h2p, b2r, dega[:N], degb[:N])
    return out


# speedup vs baseline: 10.3193x; 10.3193x over previous
import functools

import jax
import jax.numpy as jnp
from jax import lax
from jax.experimental import pallas as pl
from jax.experimental.pallas import tpu as pltpu
from jax.experimental.pallas import tpu_sc as plsc

N = 10000
D = 128
H = 128

NW = 32
K = 128
NPAD = 10240
RPW = NPAD // 16
DEGW = 16


def _deg_kernel(dst_hbm, ones_hbm, zeros_hbm, out_hbm, dst_v, ones_v, sem,
                acc_sh, *, n_chunks):
    c = lax.axis_index("c")
    s = lax.axis_index("s")
    wid = c * 16 + s
    pltpu.sync_copy(ones_hbm, ones_v)
    pltpu.sync_copy(zeros_hbm, acc_sh.at[pl.ds(s * RPW, RPW)])
    plsc.subcore_barrier()

    base = wid * (n_chunks * K)

    def body(i, carry):
        off = base + i * K
        pltpu.sync_copy(dst_hbm.at[pl.ds(off, K)], dst_v)
        pltpu.sync_copy(ones_v, acc_sh.at[dst_v], add=True)
        return carry

    lax.fori_loop(0, n_chunks, body, 0)
    plsc.subcore_barrier()
    pltpu.sync_copy(acc_sh.at[pl.ds(s * RPW, RPW)],
                    out_hbm.at[c, pl.ds(s * RPW, RPW)])


def _agg_kernel(h_hbm, src_hbm, dst_hbm, zeros_hbm, out_hbm,
                src_v, dst_v, rows_v, sem, acc_sh, *, n_chunks):
    c = lax.axis_index("c")
    s = lax.axis_index("s")
    wid = c * 16 + s
    pltpu.sync_copy(zeros_hbm, acc_sh.at[pl.ds(s * RPW, RPW)])
    plsc.subcore_barrier()

    base = wid * (n_chunks * K)

    def body(i, carry):
        off = base + i * K
        pltpu.sync_copy(src_hbm.at[pl.ds(off, K)], src_v)
        pltpu.sync_copy(dst_hbm.at[pl.ds(off, K)], dst_v)
        pltpu.async_copy(h_hbm.at[src_v], rows_v, sem).wait()
        pltpu.sync_copy(rows_v, acc_sh.at[dst_v], add=True)
        return carry

    lax.fori_loop(0, n_chunks, body, 0)
    plsc.subcore_barrier()
    pltpu.sync_copy(acc_sh.at[pl.ds(s * RPW, RPW)],
                    out_hbm.at[c, pl.ds(s * RPW, RPW)])


def _make_deg(n_chunks):
    mesh = plsc.VectorSubcoreMesh(core_axis_name="c", subcore_axis_name="s")
    return pl.kernel(
        functools.partial(_deg_kernel, n_chunks=n_chunks),
        out_type=jax.ShapeDtypeStruct((2, NPAD, DEGW), jnp.float32),
        mesh=mesh,
        scratch_types=[
            pltpu.VMEM((K,), jnp.int32),
            pltpu.VMEM((K, DEGW), jnp.float32),
            pltpu.SemaphoreType.DMA,
            pltpu.MemorySpace.VMEM_SHARED((NPAD, DEGW), jnp.float32),
        ],
    )


def _make_agg(n_chunks):
    mesh = plsc.VectorSubcoreMesh(core_axis_name="c", subcore_axis_name="s")
    return pl.kernel(
        functools.partial(_agg_kernel, n_chunks=n_chunks),
        out_type=jax.ShapeDtypeStruct((2, NPAD, H), jnp.float32),
        mesh=mesh,
        scratch_types=[
            pltpu.VMEM((K,), jnp.int32),
            pltpu.VMEM((K,), jnp.int32),
            pltpu.VMEM((K, H), jnp.float32),
            pltpu.SemaphoreType.DMA,
            pltpu.MemorySpace.VMEM_SHARED((NPAD, H), jnp.float32),
        ],
    )


RB = 2000


def _dinv(dega_ref, degb_ref):
    deg = dega_ref[:, 0:1] + degb_ref[:, 0:1] + 1.0
    return lax.rsqrt(deg)


def _mm_scale_body(x_ref, w_ref, dega_ref, degb_ref, o_ref):
    dinv = _dinv(dega_ref, degb_ref)
    h = jnp.dot(x_ref[...], w_ref[...], preferred_element_type=jnp.float32)
    o_ref[...] = h * dinv


def _mid_body(agg_ref, hp_ref, w_ref, b_ref, dega_ref, degb_ref, o_ref):
    dinv = _dinv(dega_ref, degb_ref)
    tot = agg_ref[0] + agg_ref[1] + hp_ref[...]
    out1 = jnp.maximum(tot * dinv + b_ref[...], 0.0)
    o_ref[...] = jnp.dot(out1, w_ref[...],
                         preferred_element_type=jnp.float32) * dinv


def _final_body(agg_ref, hp_ref, b_ref, dega_ref, degb_ref, o_ref):
    dinv = _dinv(dega_ref, degb_ref)
    tot = agg_ref[0] + agg_ref[1] + hp_ref[...]
    o_ref[...] = jnp.maximum(tot * dinv + b_ref[...], 0.0)


def _row_spec(width):
    return pl.BlockSpec((RB, width), lambda i: (i, 0))


def _full_spec(shape):
    return pl.BlockSpec(shape, lambda i: tuple(0 for _ in shape))


def _mm_scale(x, w, dega, degb):
    return pl.pallas_call(
        _mm_scale_body,
        grid=(N // RB,),
        in_specs=[
            _row_spec(D),
            _full_spec((D, H)),
            _row_spec(DEGW),
            _row_spec(DEGW),
        ],
        out_specs=_row_spec(H),
        out_shape=jax.ShapeDtypeStruct((N, H), jnp.float32),
    )(x, w, dega, degb)


def _mid(agg, hp, w, b, dega, degb):
    return pl.pallas_call(
        _mid_body,
        grid=(N // RB,),
        in_specs=[
            pl.BlockSpec((2, RB, H), lambda i: (0, i, 0)),
            _row_spec(H),
            _full_spec((H, H)),
            _full_spec((1, H)),
            _row_spec(DEGW),
            _row_spec(DEGW),
        ],
        out_specs=_row_spec(H),
        out_shape=jax.ShapeDtypeStruct((N, H), jnp.float32),
    )(agg, hp, w, b, dega, degb)


def _final(agg, hp, b, dega, degb):
    return pl.pallas_call(
        _final_body,
        grid=(N // RB,),
        in_specs=[
            pl.BlockSpec((2, RB, H), lambda i: (0, i, 0)),
            _row_spec(H),
            _full_spec((1, H)),
            _row_spec(DEGW),
            _row_spec(DEGW),
        ],
        out_specs=_row_spec(H),
        out_shape=jax.ShapeDtypeStruct((N, H), jnp.float32),
    )(agg, hp, b, dega, degb)


def kernel(x, edge_index, W1, b1, W2, b2):
    src = edge_index[0]
    dst = edge_index[1]
    e = src.shape[0]

    epad = -(-e // (NW * K)) * (NW * K)
    n_chunks = epad // (NW * K)
    pad = epad - e
    src_p = jnp.concatenate([src, jnp.zeros((pad,), jnp.int32)])
    dst_p = jnp.concatenate([dst, jnp.full((pad,), N, jnp.int32)])

    zeros_rows = jnp.zeros((RPW, H), jnp.float32)
    zeros_deg = jnp.zeros((RPW, DEGW), jnp.float32)
    ones_v = jnp.ones((K, DEGW), jnp.float32)

    deg = _make_deg(n_chunks)(dst_p, ones_v, zeros_deg)
    dega = deg[0]
    degb = deg[1]

    b1r = b1.reshape(1, H)
    b2r = b2.reshape(1, H)

    h1p = _mm_scale(x, W1, dega[:N], degb[:N])
    agg1 = _make_agg(n_chunks)(h1p, src_p, dst_p, zeros_rows)
    h2p = _mid(agg1[:, :N], h1p, W2, b1r, dega[:N], degb[:N])
    agg2 = _make_agg(n_chunks)(h2p, src_p, dst_p, zeros_rows)
    out = _final(agg2[:, :N], h2p, b2r, dega[:N], degb[:N])
    return out
